# baseline (speedup 1.0000x reference)
import functools

import jax
import jax.numpy as jnp
from jax import lax
from jax.experimental import pallas as pl
from jax.experimental.pallas import tpu as pltpu
from jax.experimental.pallas import tpu_sc as plsc

_NC = 2
_NS = 16
_NW = _NC * _NS


@functools.cache
def _build_sc_gather(B, D):
    rows_per_w = B // _NW
    CHUNK = 32
    n_chunks = rows_per_w // CHUNK
    NBUF = 3

    mesh = plsc.VectorSubcoreMesh(core_axis_name="c", subcore_axis_name="s")

    @functools.partial(
        pl.kernel,
        out_type=jax.ShapeDtypeStruct((B, D), jnp.float32),
        mesh=mesh,
        scratch_types=[
            pltpu.VMEM((rows_per_w,), jnp.int32),
            pltpu.VMEM((NBUF, CHUNK, D), jnp.float32),
            pltpu.SemaphoreType.DMA((NBUF,)),
            pltpu.SemaphoreType.DMA((NBUF,)),
        ],
    )
    def sc_gather(src_ref, emb_ref, out_ref, idx_v, x_buf, gsems, osems):
        wid = lax.axis_index("s") * _NC + lax.axis_index("c")
        base = wid * rows_per_w
        pltpu.sync_copy(src_ref.at[pl.ds(base, rows_per_w)], idx_v)

        gdescs = [None] * NBUF
        odescs = [None] * NBUF
        for c in range(n_chunks + 1):
            if c < n_chunks:
                b = c % NBUF
                if c >= NBUF:
                    odescs[b].wait()
                gdescs[b] = pltpu.async_copy(
                    emb_ref.at[idx_v.at[pl.ds(c * CHUNK, CHUNK)]],
                    x_buf.at[b], gsems.at[b])
            if c >= 1:
                p = (c - 1) % NBUF
                gdescs[p].wait()
                odescs[p] = pltpu.async_copy(
                    x_buf.at[p],
                    out_ref.at[pl.ds(base + (c - 1) * CHUNK, CHUNK)],
                    osems.at[p])
        for c in range(max(0, n_chunks - NBUF + 1), n_chunks):
            odescs[c % NBUF].wait()

    return sc_gather


@functools.cache
def _build_tc_ln(S, N, D, eps):
    SB = 512
    grid = (S // SB,)

    def tc_ln(tok_ref, tt_ref, pos_ref, ttab_ref, g_ref, b_ref, out_ref):
        x = tok_ref[...]
        x = x + pos_ref[...][:, None, :]
        w = tt_ref[...].astype(jnp.float32)[..., None]
        t0 = ttab_ref[0]
        t1 = ttab_ref[1]
        x = x + t0[None, None, :] + w * (t1 - t0)[None, None, :]
        mean = jnp.mean(x, axis=-1, keepdims=True)
        xc = x - mean
        var = jnp.mean(xc * xc, axis=-1, keepdims=True)
        out_ref[...] = (xc * lax.rsqrt(var + eps) * g_ref[0][None, None, :]
                        + b_ref[0][None, None, :])

    return pl.pallas_call(
        tc_ln,
        grid=grid,
        in_specs=[
            pl.BlockSpec((SB, N, D), lambda i: (i, 0, 0)),
            pl.BlockSpec((SB, N), lambda i: (i, 0)),
            pl.BlockSpec((SB, D), lambda i: (i, 0)),
            pl.BlockSpec((2, D), lambda i: (0, 0)),
            pl.BlockSpec((1, D), lambda i: (0, 0)),
            pl.BlockSpec((1, D), lambda i: (0, 0)),
        ],
        out_specs=pl.BlockSpec((SB, N, D), lambda i: (i, 0, 0)),
        out_shape=jax.ShapeDtypeStruct((S, N, D), jnp.float32),
    )


def kernel(src, token_type_input, embed_table, pos_table, tok_type_table,
           ln_gamma, ln_beta):
    S, N = src.shape
    D = embed_table.shape[1]
    B = S * N
    tok = _build_sc_gather(B, D)(src.reshape(B).astype(jnp.int32),
                                 embed_table)
    out = _build_tc_ln(S, N, D, 1e-5)(
        tok.reshape(S, N, D),
        token_type_input.astype(jnp.int32),
        pos_table,
        tok_type_table,
        ln_gamma.reshape(1, D),
        ln_beta.reshape(1, D),
    )
    return out


# speedup vs baseline: 1.0155x; 1.0155x over previous
import functools

import jax
import jax.numpy as jnp
from jax import lax
from jax.experimental import pallas as pl
from jax.experimental.pallas import tpu as pltpu
from jax.experimental.pallas import tpu_sc as plsc

_NC = 2
_NS = 16
_NW = _NC * _NS


@functools.cache
def _build_sc_gather(B, D):
    rows_per_w = B // _NW
    CHUNK = 16
    n_chunks = rows_per_w // CHUNK
    NBUF = 4

    mesh = plsc.VectorSubcoreMesh(core_axis_name="c", subcore_axis_name="s")

    @functools.partial(
        pl.kernel,
        out_type=jax.ShapeDtypeStruct((B, D), jnp.float32),
        mesh=mesh,
        scratch_types=[
            pltpu.VMEM((rows_per_w,), jnp.int32),
            pltpu.VMEM((NBUF, CHUNK, D), jnp.float32),
            pltpu.SemaphoreType.DMA((NBUF,)),
            pltpu.SemaphoreType.DMA((NBUF,)),
        ],
    )
    def sc_gather(src_ref, emb_ref, out_ref, idx_v, x_buf, gsems, osems):
        wid = lax.axis_index("s") * _NC + lax.axis_index("c")
        base = wid * rows_per_w
        pltpu.sync_copy(src_ref.at[pl.ds(base, rows_per_w)], idx_v)

        gdescs = [None] * NBUF
        odescs = [None] * NBUF
        for c in range(n_chunks + 1):
            if c < n_chunks:
                b = c % NBUF
                if c >= NBUF:
                    odescs[b].wait()
                gdescs[b] = pltpu.async_copy(
                    emb_ref.at[idx_v.at[pl.ds(c * CHUNK, CHUNK)]],
                    x_buf.at[b], gsems.at[b])
            if c >= 1:
                p = (c - 1) % NBUF
                gdescs[p].wait()
                odescs[p] = pltpu.async_copy(
                    x_buf.at[p],
                    out_ref.at[pl.ds(base + (c - 1) * CHUNK, CHUNK)],
                    osems.at[p])
        for c in range(max(0, n_chunks - NBUF + 1), n_chunks):
            odescs[c % NBUF].wait()

    return sc_gather


@functools.cache
def _build_tc_ln(S, N, D, eps):
    SB = 256
    grid = (S // SB,)

    def tc_ln(tok_ref, tt_ref, pos_ref, ttab_ref, g_ref, b_ref, out_ref):
        x = tok_ref[...]
        x = x + pos_ref[...][:, None, :]
        w = tt_ref[...].astype(jnp.float32)[..., None]
        t0 = ttab_ref[0]
        t1 = ttab_ref[1]
        x = x + t0[None, None, :] + w * (t1 - t0)[None, None, :]
        mean = jnp.mean(x, axis=-1, keepdims=True)
        xc = x - mean
        var = jnp.mean(xc * xc, axis=-1, keepdims=True)
        out_ref[...] = (xc * lax.rsqrt(var + eps) * g_ref[0][None, None, :]
                        + b_ref[0][None, None, :])

    return pl.pallas_call(
        tc_ln,
        grid=grid,
        in_specs=[
            pl.BlockSpec((SB, N, D), lambda i: (i, 0, 0)),
            pl.BlockSpec((SB, N), lambda i: (i, 0)),
            pl.BlockSpec((SB, D), lambda i: (i, 0)),
            pl.BlockSpec((2, D), lambda i: (0, 0)),
            pl.BlockSpec((1, D), lambda i: (0, 0)),
            pl.BlockSpec((1, D), lambda i: (0, 0)),
        ],
        out_specs=pl.BlockSpec((SB, N, D), lambda i: (i, 0, 0)),
        out_shape=jax.ShapeDtypeStruct((S, N, D), jnp.float32),
    )


def kernel(src, token_type_input, embed_table, pos_table, tok_type_table,
           ln_gamma, ln_beta):
    S, N = src.shape
    D = embed_table.shape[1]
    B = S * N
    tok = _build_sc_gather(B, D)(src.reshape(B).astype(jnp.int32),
                                 embed_table)
    out = _build_tc_ln(S, N, D, 1e-5)(
        tok.reshape(S, N, D),
        token_type_input.astype(jnp.int32),
        pos_table,
        tok_type_table,
        ln_gamma.reshape(1, D),
        ln_beta.reshape(1, D),
    )
    return out
